# baseline (speedup 1.0000x reference)
import functools

import jax
import jax.numpy as jnp
from jax import lax
from jax.experimental import pallas as pl
from jax.experimental.pallas import tpu as pltpu
from jax.experimental.pallas import tpu_sc as plsc

_K = 100
_E = 2


@functools.cache
def _build(N, S, V, D):
    info = plsc.get_sparse_core_info()
    NC, NS = info.num_cores, info.num_subcores
    NW = NC * NS
    assert N % (NW * _E) == 0 and S == 2 * _K
    e_per_w = N // NW
    steps = e_per_w // _E
    kidx_per_w = e_per_w * S // _K

    mesh = plsc.VectorSubcoreMesh(core_axis_name="c", subcore_axis_name="s")

    @functools.partial(
        pl.kernel,
        out_type=jax.ShapeDtypeStruct((N, S, D), jnp.float32),
        mesh=mesh,
        scratch_types=[
            pltpu.VMEM((kidx_per_w, _K), jnp.int32),
            pltpu.VMEM((_E, S, D), jnp.float32),
            pltpu.VMEM((_E, S, D), jnp.float32),
            pltpu.SemaphoreType.DMA,
            pltpu.SemaphoreType.DMA,
        ],
        compiler_params=pltpu.CompilerParams(use_tc_tiling_on_sc=False),
    )
    def gather_kernel(idx_hbm, table_hbm, out_hbm, idx_v, b0, b1, s0, s1):
        wid = lax.axis_index("s") * NC + lax.axis_index("c")
        ebase = wid * e_per_w
        pltpu.sync_copy(idx_hbm.at[pl.ds(wid * kidx_per_w, kidx_per_w)],
                        idx_v)

        def fire(chunk, buf, sem):
            for e in range(_E):
                for h in range(2):
                    pltpu.async_copy(
                        table_hbm.at[idx_v.at[chunk * 2 * _E + 2 * e + h]],
                        buf.at[e, pl.ds(h * _K, _K)], sem)

        def drain(buf, sem):
            pltpu.make_async_copy(out_hbm.at[pl.ds(0, _E)], buf, sem).wait()

        fire(0, b0, s0)

        @pl.loop(0, steps, step=2)
        def _(g):
            fire(g + 1, b1, s1)
            drain(b0, s0)
            pltpu.sync_copy(b0, out_hbm.at[pl.ds(ebase + g * _E, _E)])

            @pl.when(g + 2 < steps)
            def _():
                fire(g + 2, b0, s0)

            drain(b1, s1)
            pltpu.sync_copy(b1,
                            out_hbm.at[pl.ds(ebase + (g + 1) * _E, _E)])

    return gather_kernel


def kernel(idx, pe):
    N, S = idx.shape
    V, D = pe.shape
    idx_rows = idx.reshape(N * S // _K, _K).astype(jnp.int32)
    return _build(N, S, V, D)(idx_rows, pe)
